# Initial kernel scaffold; baseline (speedup 1.0000x reference)
import jax
import jax.numpy as jnp
from jax.experimental import pallas as pl


def kernel(x, edge_index, W1, b1, W2, b2, Wl, bl):
    raise NotImplementedError("write your pallas kernel here")


# speedup vs baseline: 4.9318x; 4.9318x over previous
import functools
import jax
import jax.numpy as jnp
from jax import lax
from jax.experimental import pallas as pl
from jax.experimental.pallas import tpu as pltpu
from jax.experimental.pallas import tpu_sc as plsc

NC, NS = 2, 16
NW = NC * NS
N, E = 10000, 320000
H = 256
FH = 128
HALF = N // NC
DUMP = HALF
ACC_ROWS = HALF + 128
CHUNK = 80
WB = 320

_mesh = plsc.VectorSubcoreMesh(core_axis_name="c", subcore_axis_name="s")
_sc_params = pltpu.CompilerParams(needs_layout_passes=False)


@functools.partial(
    pl.kernel,
    mesh=_mesh,
    out_type=jax.ShapeDtypeStruct((NW, N), jnp.float32),
    scratch_types=[
        pltpu.VMEM((E // NW,), jnp.int32),
        pltpu.VMEM((N,), jnp.float32),
    ],
    compiler_params=_sc_params,
)
def _sc_degree(dst_hbm, out_hbm, dstv, hist):
    wid = lax.axis_index("s") * NC + lax.axis_index("c")
    epw = E // NW
    pltpu.sync_copy(dst_hbm.at[pl.ds(wid * epw, epw)], dstv)
    zeros16 = jnp.zeros((16,), jnp.float32)

    def zero_body(i, carry):
        hist[pl.ds(i * 16, 16)] = zeros16
        return carry

    lax.fori_loop(0, N // 16, zero_body, 0)
    ones16 = jnp.ones((16,), jnp.float32)

    def acc_body(i, carry):
        idx = dstv[pl.ds(i * 16, 16)]
        plsc.addupdate_scatter(hist, [idx], ones16)
        return carry

    lax.fori_loop(0, epw // 16, acc_body, 0)
    pltpu.sync_copy(hist, out_hbm.at[wid])


@functools.partial(
    pl.kernel,
    mesh=_mesh,
    out_type=jax.ShapeDtypeStruct((N, FH), jnp.float32),
    scratch_types=[
        pltpu.VMEM((CHUNK,), jnp.int32),
        pltpu.VMEM((CHUNK,), jnp.int32),
        pltpu.VMEM((CHUNK, FH), jnp.float32),
        pltpu.VMEM((WB, FH), jnp.float32),
        pltpu.VMEM_SHARED((ACC_ROWS, FH), jnp.float32),
        pltpu.SemaphoreType.DMA,
    ],
    compiler_params=_sc_params,
)
def _sc_scatter(zh_hbm, src_hbm, dst_hbm, out_hbm, srcv, dstv, rows, zbuf,
                accum, sem):
    c = lax.axis_index("c")
    s = lax.axis_index("s")
    zeros16 = jnp.zeros((16,), jnp.float32)

    def zero_body(r, carry):
        for k in range(FH // 16):
            zbuf[r, pl.ds(k * 16, 16)] = zeros16
        return carry

    lax.fori_loop(0, WB, zero_body, 0)
    start = pl.multiple_of(jnp.minimum(s * WB, HALF - WB), 8)
    pltpu.sync_copy(zbuf, accum.at[pl.ds(start, WB)])
    plsc.subcore_barrier()

    epw = E // NS
    base0 = s * epw
    cbase = c * HALF

    def edge_body(i, carry):
        b = base0 + i * CHUNK
        pltpu.sync_copy(src_hbm.at[pl.ds(b, CHUNK)], srcv)
        pltpu.sync_copy(dst_hbm.at[pl.ds(b, CHUNK)], dstv)

        def fix_body(j, cc):
            d16 = dstv[pl.ds(j * 16, 16)]
            rel = d16 - cbase
            oob = (rel < 0) | (rel >= HALF)
            dstv[pl.ds(j * 16, 16)] = jnp.where(oob, DUMP, rel)
            return cc

        lax.fori_loop(0, CHUNK // 16, fix_body, 0)
        pltpu.async_copy(zh_hbm.at[srcv], rows, sem).wait()
        pltpu.sync_copy(rows, accum.at[dstv], add=True)
        return carry

    lax.fori_loop(0, epw // CHUNK, edge_body, 0)
    plsc.subcore_barrier()
    pltpu.sync_copy(accum.at[pl.ds(start, WB)],
                    out_hbm.at[pl.ds(cbase + start, WB)])


_BLK = 1000


def _tc0_body(degp_ref, dinv_ref):
    deg = jnp.sum(degp_ref[...], axis=0, keepdims=True) + 1.0
    dinv_ref[...] = lax.rsqrt(deg)


def _tc1_body(x_ref, w_ref, dinv_ref, za_ref, zb_ref):
    dinv = dinv_ref[...]
    xw = jnp.dot(x_ref[...], w_ref[...], preferred_element_type=jnp.float32)
    z = xw * dinv
    za_ref[...] = z[:, :FH]
    zb_ref[...] = z[:, FH:]


def _mid_h(sa, sb, za, zb, b_ref, dinv):
    agg_a = sa[...] + za[...]
    agg_b = sb[...] + zb[...]
    h = jnp.concatenate([agg_a, agg_b], axis=1) * dinv + b_ref[...]
    return jnp.maximum(h, 0.0)


def _tc2_body(sa, sb, za, zb, dinv_ref, w_ref, b_ref, za2_ref, zb2_ref):
    dinv = dinv_ref[...]
    h = _mid_h(sa, sb, za, zb, b_ref, dinv)
    znew = jnp.dot(h, w_ref[...], preferred_element_type=jnp.float32)
    znew = znew * dinv
    za2_ref[...] = znew[:, :FH]
    zb2_ref[...] = znew[:, FH:]


def _tc3_body(sa, sb, za, zb, dinv_ref, b2_ref, wl_ref, bl_ref, out_ref):
    dinv = dinv_ref[...]
    h = _mid_h(sa, sb, za, zb, b2_ref, dinv)
    out_ref[...] = (
        jnp.dot(h, wl_ref[...], preferred_element_type=jnp.float32)
        + bl_ref[...]
    )


def _row_spec(width):
    return pl.BlockSpec((_BLK, width), lambda i: (i, 0))


def _full_spec(shape):
    nd = len(shape)
    return pl.BlockSpec(shape, lambda i: (0,) * nd)


def _half_shapes():
    return [jax.ShapeDtypeStruct((N, FH), jnp.float32) for _ in range(2)]


def kernel(x, edge_index, W1, b1, W2, b2, Wl, bl):
    src = edge_index[0]
    dst = edge_index[1]
    C = Wl.shape[1]
    grid = (N // _BLK,)

    degp = _sc_degree(dst)

    dinv_row = pl.pallas_call(
        _tc0_body,
        in_specs=[pl.BlockSpec((NW, N), lambda: (0, 0))],
        out_specs=pl.BlockSpec((1, N), lambda: (0, 0)),
        out_shape=jax.ShapeDtypeStruct((1, N), jnp.float32),
    )(degp)
    dinv = dinv_row.reshape(N, 1)

    za, zb = pl.pallas_call(
        _tc1_body,
        grid=grid,
        in_specs=[
            _row_spec(x.shape[1]),
            _full_spec(W1.shape),
            _row_spec(1),
        ],
        out_specs=[_row_spec(FH)] * 2,
        out_shape=_half_shapes(),
    )(x, W1, dinv)

    sa = _sc_scatter(za, src, dst)
    sb = _sc_scatter(zb, src, dst)

    za2, zb2 = pl.pallas_call(
        _tc2_body,
        grid=grid,
        in_specs=[_row_spec(FH)] * 4
        + [_row_spec(1), _full_spec(W2.shape), _full_spec((1, H))],
        out_specs=[_row_spec(FH)] * 2,
        out_shape=_half_shapes(),
    )(sa, sb, za, zb, dinv, W2, b1.reshape(1, H))

    sa2 = _sc_scatter(za2, src, dst)
    sb2 = _sc_scatter(zb2, src, dst)

    out = pl.pallas_call(
        _tc3_body,
        grid=grid,
        in_specs=[_row_spec(FH)] * 4
        + [_row_spec(1), _full_spec((1, H)), _full_spec(Wl.shape),
           _full_spec((1, C))],
        out_specs=_row_spec(C),
        out_shape=jax.ShapeDtypeStruct((N, C), jnp.float32),
    )(sa2, sb2, za2, zb2, dinv, b2.reshape(1, H), Wl, bl.reshape(1, C))

    return out


# speedup vs baseline: 10.7832x; 2.1865x over previous
import functools
import jax
import jax.numpy as jnp
from jax import lax
from jax.experimental import pallas as pl
from jax.experimental.pallas import tpu as pltpu
from jax.experimental.pallas import tpu_sc as plsc

NC, NS = 2, 16
NW = NC * NS
N, E = 10000, 320000
H = 256
FH = 128
HALF = N // NC
DUMP = HALF
ACC_ROWS = HALF + 8
CHUNK = 80
WB = 320

_mesh = plsc.VectorSubcoreMesh(core_axis_name="c", subcore_axis_name="s")
_sc_params = pltpu.CompilerParams(needs_layout_passes=False)


@functools.partial(
    pl.kernel,
    mesh=_mesh,
    out_type=jax.ShapeDtypeStruct((NW, N), jnp.float32),
    scratch_types=[
        pltpu.VMEM((E // NW,), jnp.int32),
        pltpu.VMEM((N,), jnp.float32),
    ],
    compiler_params=_sc_params,
)
def _sc_degree(dst_hbm, out_hbm, dstv, hist):
    wid = lax.axis_index("s") * NC + lax.axis_index("c")
    epw = E // NW
    pltpu.sync_copy(dst_hbm.at[pl.ds(wid * epw, epw)], dstv)
    zeros16 = jnp.zeros((16,), jnp.float32)

    def zero_body(i, carry):
        hist[pl.ds(i * 16, 16)] = zeros16
        return carry

    lax.fori_loop(0, N // 16, zero_body, 0)
    ones16 = jnp.ones((16,), jnp.float32)

    def acc_body(i, carry):
        idx = dstv[pl.ds(i * 16, 16)]
        plsc.addupdate_scatter(hist, [idx], ones16)
        return carry

    lax.fori_loop(0, epw // 16, acc_body, 0)
    pltpu.sync_copy(hist, out_hbm.at[wid])


_EPW = E // NS
_P = 5
_EPP = _EPW // _P
_NCP = _EPP // CHUNK
_ZB = 80


@functools.partial(
    pl.kernel,
    mesh=_mesh,
    out_type=jax.ShapeDtypeStruct((N, FH), jnp.float32),
    scratch_types=[
        pltpu.VMEM((_EPP,), jnp.int32),
        pltpu.VMEM((_EPP,), jnp.int32),
        pltpu.VMEM((_NCP, CHUNK), jnp.int32),
        pltpu.VMEM((CHUNK, FH), jnp.float32),
        pltpu.VMEM((CHUNK, FH), jnp.float32),
        pltpu.VMEM((_ZB, FH), jnp.float32),
        pltpu.VMEM_SHARED((ACC_ROWS, FH), jnp.float32),
        pltpu.SemaphoreType.DMA,
        pltpu.SemaphoreType.DMA,
    ],
    compiler_params=_sc_params,
)
def _sc_scatter(zh_hbm, src_hbm, dst_hbm, out_hbm, src1d, dst1d, dst2d,
                rows0, rows1, zbuf, accum, sem0, sem1):
    c = lax.axis_index("c")
    s = lax.axis_index("s")
    zeros16 = jnp.zeros((16,), jnp.float32)

    def zero_body(r, carry):
        for k in range(FH // 16):
            zbuf[r, pl.ds(k * 16, 16)] = zeros16
        return carry

    lax.fori_loop(0, _ZB, zero_body, 0)
    start = pl.multiple_of(jnp.minimum(s * WB, HALF - WB), 8)
    for j in range(WB // _ZB):
        pltpu.sync_copy(zbuf, accum.at[pl.ds(start + j * _ZB, _ZB)])
    plsc.subcore_barrier()

    cbase = c * HALF
    rows = (rows0, rows1)
    sems = (sem0, sem1)

    for p in range(_P):
        base = s * _EPW + p * _EPP
        pltpu.sync_copy(src_hbm.at[pl.ds(base, _EPP)], src1d)
        pltpu.sync_copy(dst_hbm.at[pl.ds(base, _EPP)], dst1d)

        def remap_body(i, carry):
            for k in range(CHUNK // 16):
                d16 = dst1d[pl.ds(i * CHUNK + k * 16, 16)]
                rel = d16 - cbase
                oob = (rel < 0) | (rel >= HALF)
                dst2d[i, pl.ds(k * 16, 16)] = jnp.where(oob, DUMP, rel)
            return carry

        lax.fori_loop(0, _NCP, remap_body, 0)

        pltpu.async_copy(zh_hbm.at[src1d.at[pl.ds(0, CHUNK)]], rows0, sem0)
        pltpu.async_copy(zh_hbm.at[src1d.at[pl.ds(CHUNK, CHUNK)]], rows1,
                         sem1)

        def group_body(g, carry):
            for j in range(2):
                i = g * 2 + j
                pltpu.make_async_copy(
                    zh_hbm.at[src1d.at[pl.ds(i * CHUNK, CHUNK)]], rows[j],
                    sems[j]).wait()
                pltpu.sync_copy(rows[j], accum.at[dst2d.at[i]], add=True)
                nxt = i + 2

                @pl.when(nxt < _NCP)
                def _():
                    pltpu.async_copy(
                        zh_hbm.at[src1d.at[pl.ds(nxt * CHUNK, CHUNK)]],
                        rows[j], sems[j])
            return carry

        lax.fori_loop(0, _NCP // 2, group_body, 0)

    plsc.subcore_barrier()
    pltpu.sync_copy(accum.at[pl.ds(start, WB)],
                    out_hbm.at[pl.ds(cbase + start, WB)])


_BLK = 1000


def _tc0_body(degp_ref, dinv_ref):
    deg = jnp.sum(degp_ref[...], axis=0, keepdims=True) + 1.0
    dinv_ref[...] = lax.rsqrt(deg)


def _tc1_body(x_ref, w_ref, dinv_ref, za_ref, zb_ref):
    dinv = dinv_ref[...]
    xw = jnp.dot(x_ref[...], w_ref[...], preferred_element_type=jnp.float32)
    z = xw * dinv
    za_ref[...] = z[:, :FH]
    zb_ref[...] = z[:, FH:]


def _mid_h(sa, sb, za, zb, b_ref, dinv):
    agg_a = sa[...] + za[...]
    agg_b = sb[...] + zb[...]
    h = jnp.concatenate([agg_a, agg_b], axis=1) * dinv + b_ref[...]
    return jnp.maximum(h, 0.0)


def _tc2_body(sa, sb, za, zb, dinv_ref, w_ref, b_ref, za2_ref, zb2_ref):
    dinv = dinv_ref[...]
    h = _mid_h(sa, sb, za, zb, b_ref, dinv)
    znew = jnp.dot(h, w_ref[...], preferred_element_type=jnp.float32)
    znew = znew * dinv
    za2_ref[...] = znew[:, :FH]
    zb2_ref[...] = znew[:, FH:]


def _tc3_body(sa, sb, za, zb, dinv_ref, b2_ref, wl_ref, bl_ref, out_ref):
    dinv = dinv_ref[...]
    h = _mid_h(sa, sb, za, zb, b2_ref, dinv)
    out_ref[...] = (
        jnp.dot(h, wl_ref[...], preferred_element_type=jnp.float32)
        + bl_ref[...]
    )


def _row_spec(width):
    return pl.BlockSpec((_BLK, width), lambda i: (i, 0))


def _full_spec(shape):
    nd = len(shape)
    return pl.BlockSpec(shape, lambda i: (0,) * nd)


def _half_shapes():
    return [jax.ShapeDtypeStruct((N, FH), jnp.float32) for _ in range(2)]


def kernel(x, edge_index, W1, b1, W2, b2, Wl, bl):
    src = edge_index[0]
    dst = edge_index[1]
    C = Wl.shape[1]
    grid = (N // _BLK,)

    degp = _sc_degree(dst)

    dinv_row = pl.pallas_call(
        _tc0_body,
        in_specs=[pl.BlockSpec((NW, N), lambda: (0, 0))],
        out_specs=pl.BlockSpec((1, N), lambda: (0, 0)),
        out_shape=jax.ShapeDtypeStruct((1, N), jnp.float32),
    )(degp)
    dinv = dinv_row.reshape(N, 1)

    za, zb = pl.pallas_call(
        _tc1_body,
        grid=grid,
        in_specs=[
            _row_spec(x.shape[1]),
            _full_spec(W1.shape),
            _row_spec(1),
        ],
        out_specs=[_row_spec(FH)] * 2,
        out_shape=_half_shapes(),
    )(x, W1, dinv)

    sa = _sc_scatter(za, src, dst)
    sb = _sc_scatter(zb, src, dst)

    za2, zb2 = pl.pallas_call(
        _tc2_body,
        grid=grid,
        in_specs=[_row_spec(FH)] * 4
        + [_row_spec(1), _full_spec(W2.shape), _full_spec((1, H))],
        out_specs=[_row_spec(FH)] * 2,
        out_shape=_half_shapes(),
    )(sa, sb, za, zb, dinv, W2, b1.reshape(1, H))

    sa2 = _sc_scatter(za2, src, dst)
    sb2 = _sc_scatter(zb2, src, dst)

    out = pl.pallas_call(
        _tc3_body,
        grid=grid,
        in_specs=[_row_spec(FH)] * 4
        + [_row_spec(1), _full_spec((1, H)), _full_spec(Wl.shape),
           _full_spec((1, C))],
        out_specs=_row_spec(C),
        out_shape=jax.ShapeDtypeStruct((N, C), jnp.float32),
    )(sa2, sb2, za2, zb2, dinv, b2.reshape(1, H), Wl, bl.reshape(1, C))

    return out
